# baseline (speedup 1.0000x reference)
import functools

import jax
import jax.numpy as jnp
from jax import lax
from jax.experimental import pallas as pl
from jax.experimental.pallas import tpu as pltpu
from jax.experimental.pallas import tpu_sc as plsc

_NW = 32
_CH = 128


@functools.lru_cache(maxsize=None)
def _make_tc_pack(V, D, C=16384):
    off_blocks = (V // 2) // C
    H = V - off_blocks * C
    G = (H + C - 1) // C

    def body(lo_ref, hi_ref, out_ref):
        out_ref[:, :D] = lo_ref[...].T
        out_ref[:, D:] = hi_ref[...].T

    return pl.pallas_call(
        body,
        grid=(G,),
        in_specs=[
            pl.BlockSpec((D, C), lambda i: (0, i)),
            pl.BlockSpec((D, C), lambda i: (0, i + off_blocks)),
        ],
        out_specs=pl.BlockSpec((C, 2 * D), lambda i: (i, 0)),
        out_shape=jax.ShapeDtypeStruct((H, 2 * D), jnp.float32),
    )


@functools.lru_cache(maxsize=None)
def _make_sc_gather(V2, D2, N):
    assert N % (_NW * _CH) == 0
    nch = N // (_NW * _CH)
    mesh = plsc.VectorSubcoreMesh(core_axis_name="c", subcore_axis_name="s")

    @functools.partial(
        pl.kernel,
        mesh=mesh,
        out_type=jax.ShapeDtypeStruct((N, D2), jnp.float32),
        scratch_types=[
            pltpu.VMEM((nch, _CH), jnp.int32),
            pltpu.VMEM((_CH, D2), jnp.float32),
            pltpu.VMEM((_CH, D2), jnp.float32),
            pltpu.SemaphoreType.DMA,
            pltpu.SemaphoreType.DMA,
        ],
    )
    def sc_gather(table_hbm, idx_hbm, out_hbm, idx_v, rows_v0, rows_v1, sem0, sem1):
        wid = lax.axis_index("s") * 2 + lax.axis_index("c")
        pltpu.sync_copy(idx_hbm.at[wid], idx_v)
        base = wid * (nch * _CH)
        bufs = (rows_v0, rows_v1)
        sems = (sem0, sem1)

        pltpu.async_copy(table_hbm.at[idx_v.at[0]], bufs[0], sems[0])

        def body(g, carry):
            for b in range(2):
                ch = 2 * g + b

                @pl.when(ch + 1 < nch)
                def _():
                    pltpu.async_copy(
                        table_hbm.at[idx_v.at[ch + 1]], bufs[1 - b], sems[1 - b]
                    )

                pltpu.make_async_copy(
                    table_hbm.at[idx_v.at[ch]], bufs[b], sems[b]
                ).wait()
                pltpu.sync_copy(bufs[b], out_hbm.at[pl.ds(base + ch * _CH, _CH)])
            return carry

        lax.fori_loop(0, nch // 2, body, 0)

    return sc_gather


@functools.lru_cache(maxsize=None)
def _make_tc_fuse_ln(B, L, D, bb=32):
    assert B % bb == 0
    R = bb * L

    def body(emb_ref, sel_ref, seg_ref, pos_ref, se_ref, g_ref, b_ref, out_ref):
        g2 = emb_ref[...].reshape(bb, L, 2 * D)
        sel = sel_ref[...][:, :, None]
        s = seg_ref[...][:, :, None]
        lo = g2[:, :, :D]
        hi = g2[:, :, D:]
        e0 = se_ref[0, :][None, None, :]
        e1 = se_ref[1, :][None, None, :]
        h = lo + sel * (hi - lo) + pos_ref[...][None, :, :] + e0 + s * (e1 - e0)
        hf = h.reshape(R, D)
        J = jnp.full((D, D), 1.0 / D, dtype=jnp.float32)
        m = jnp.dot(hf, J, preferred_element_type=jnp.float32)
        c = hf - m
        v = jnp.dot(c * c, J, preferred_element_type=jnp.float32)
        o = c * lax.rsqrt(v + 1e-5) * g_ref[...] + b_ref[...]
        out_ref[...] = o.reshape(bb, L, D)

    return pl.pallas_call(
        body,
        grid=(B // bb,),
        in_specs=[
            pl.BlockSpec((R, 2 * D), lambda i: (i, 0)),
            pl.BlockSpec((bb, L), lambda i: (i, 0)),
            pl.BlockSpec((bb, L), lambda i: (i, 0)),
            pl.BlockSpec((L, D), lambda i: (0, 0)),
            pl.BlockSpec((2, D), lambda i: (0, 0)),
            pl.BlockSpec((1, D), lambda i: (0, 0)),
            pl.BlockSpec((1, D), lambda i: (0, 0)),
        ],
        out_specs=pl.BlockSpec((bb, L, D), lambda i: (i, 0, 0)),
        out_shape=jax.ShapeDtypeStruct((B, L, D), jnp.float32),
    )


def kernel(x, seg, tok_emb, pos_emb, seg_emb, gamma, beta):
    B, L = x.shape
    V, D = tok_emb.shape
    N = B * L
    x = x.astype(jnp.int32)
    off = ((V // 2) // 16384) * 16384
    sel = (x >= off).astype(jnp.float32)
    idx2 = jnp.where(x >= off, x - off, x).reshape(_NW, N // (_NW * _CH), _CH)
    segf = seg.astype(jnp.float32)
    table2 = _make_tc_pack(V, D)(tok_emb.T, tok_emb.T)
    gathered = _make_sc_gather(V - off, 2 * D, N)(table2, idx2)
    return _make_tc_fuse_ln(B, L, D)(
        gathered, sel, segf, pos_emb[:L], seg_emb,
        gamma[None, :], beta[None, :],
    )


# speedup vs baseline: 1.0046x; 1.0046x over previous
import functools

import jax
import jax.numpy as jnp
from jax import lax
from jax.experimental import pallas as pl
from jax.experimental.pallas import tpu as pltpu
from jax.experimental.pallas import tpu_sc as plsc

_NW = 32
_CH = 128


@functools.lru_cache(maxsize=None)
def _make_tc_pack(V, D, C=16384):
    off_blocks = (V // 2) // C
    H = V - off_blocks * C
    G = (H + C - 1) // C

    def body(lo_ref, hi_ref, out_ref):
        out_ref[:, :D] = lo_ref[...].T
        out_ref[:, D:] = hi_ref[...].T

    return pl.pallas_call(
        body,
        grid=(G,),
        in_specs=[
            pl.BlockSpec((D, C), lambda i: (0, i)),
            pl.BlockSpec((D, C), lambda i: (0, i + off_blocks)),
        ],
        out_specs=pl.BlockSpec((C, 2 * D), lambda i: (i, 0)),
        out_shape=jax.ShapeDtypeStruct((H, 2 * D), jnp.float32),
    )


@functools.lru_cache(maxsize=None)
def _make_sc_gather(V2, D2, N):
    assert N % (_NW * _CH) == 0
    nch = N // (_NW * _CH)
    mesh = plsc.VectorSubcoreMesh(core_axis_name="c", subcore_axis_name="s")

    @functools.partial(
        pl.kernel,
        mesh=mesh,
        out_type=jax.ShapeDtypeStruct((N, D2), jnp.float32),
        scratch_types=[
            pltpu.VMEM((nch, _CH), jnp.int32),
            pltpu.VMEM((_CH, D2), jnp.float32),
            pltpu.VMEM((_CH, D2), jnp.float32),
            pltpu.SemaphoreType.DMA,
            pltpu.SemaphoreType.DMA,
        ],
    )
    def sc_gather(table_hbm, idx_hbm, out_hbm, idx_v, rows_v0, rows_v1, sem0, sem1):
        wid = lax.axis_index("s") * 2 + lax.axis_index("c")
        pltpu.sync_copy(idx_hbm.at[wid], idx_v)
        base = wid * (nch * _CH)
        bufs = (rows_v0, rows_v1)
        sems = (sem0, sem1)

        pltpu.async_copy(table_hbm.at[idx_v.at[0]], bufs[0], sems[0])

        def body(g, carry):
            for b in range(2):
                ch = 2 * g + b

                @pl.when(ch + 1 < nch)
                def _():
                    pltpu.async_copy(
                        table_hbm.at[idx_v.at[ch + 1]], bufs[1 - b], sems[1 - b]
                    )

                pltpu.make_async_copy(
                    table_hbm.at[idx_v.at[ch]], bufs[b], sems[b]
                ).wait()
                pltpu.sync_copy(bufs[b], out_hbm.at[pl.ds(base + ch * _CH, _CH)])
            return carry

        lax.fori_loop(0, nch // 2, body, 0)

    return sc_gather


@functools.lru_cache(maxsize=None)
def _make_tc_fuse_ln(B, L, D, bb=16):
    assert B % bb == 0
    R = bb * L

    def body(emb_ref, sel_ref, seg_ref, pos_ref, se_ref, g_ref, b_ref, out_ref):
        g2 = emb_ref[...].reshape(bb, L, 2 * D)
        sel = sel_ref[...][:, :, None]
        s = seg_ref[...][:, :, None]
        lo = g2[:, :, :D]
        hi = g2[:, :, D:]
        e0 = se_ref[0, :][None, None, :]
        e1 = se_ref[1, :][None, None, :]
        h = lo + sel * (hi - lo) + pos_ref[...][None, :, :] + e0 + s * (e1 - e0)
        hf = h.reshape(R, D)
        J = jnp.full((D, D), 1.0 / D, dtype=jnp.float32)
        m = jnp.dot(hf, J, preferred_element_type=jnp.float32)
        c = hf - m
        v = jnp.dot(c * c, J, preferred_element_type=jnp.float32)
        o = c * lax.rsqrt(v + 1e-5) * g_ref[...] + b_ref[...]
        out_ref[...] = o.reshape(bb, L, D)

    return pl.pallas_call(
        body,
        grid=(B // bb,),
        in_specs=[
            pl.BlockSpec((R, 2 * D), lambda i: (i, 0)),
            pl.BlockSpec((bb, L), lambda i: (i, 0)),
            pl.BlockSpec((bb, L), lambda i: (i, 0)),
            pl.BlockSpec((L, D), lambda i: (0, 0)),
            pl.BlockSpec((2, D), lambda i: (0, 0)),
            pl.BlockSpec((1, D), lambda i: (0, 0)),
            pl.BlockSpec((1, D), lambda i: (0, 0)),
        ],
        out_specs=pl.BlockSpec((bb, L, D), lambda i: (i, 0, 0)),
        out_shape=jax.ShapeDtypeStruct((B, L, D), jnp.float32),
    )


def kernel(x, seg, tok_emb, pos_emb, seg_emb, gamma, beta):
    B, L = x.shape
    V, D = tok_emb.shape
    N = B * L
    x = x.astype(jnp.int32)
    off = ((V // 2) // 16384) * 16384
    sel = (x >= off).astype(jnp.float32)
    idx2 = jnp.where(x >= off, x - off, x).reshape(_NW, N // (_NW * _CH), _CH)
    segf = seg.astype(jnp.float32)
    table2 = _make_tc_pack(V, D)(tok_emb.T, tok_emb.T)
    gathered = _make_sc_gather(V - off, 2 * D, N)(table2, idx2)
    return _make_tc_fuse_ln(B, L, D)(
        gathered, sel, segf, pos_emb[:L], seg_emb,
        gamma[None, :], beta[None, :],
    )
